# Initial kernel scaffold; baseline (speedup 1.0000x reference)
import jax
import jax.numpy as jnp
from jax.experimental import pallas as pl


def kernel(x, W):
    raise NotImplementedError("write your pallas kernel here")


# speedup vs baseline: 1.7742x; 1.7742x over previous
import functools

import jax
import jax.numpy as jnp
from jax import lax
from jax.experimental import pallas as pl
from jax.experimental.pallas import tpu as pltpu

N_EMBD = 768
NUM_EXPERTS = 64
N_TOKENS = 32768
BLOCK_T = 1024


def _router_block(x_ref, w_ref, probs_ref, topp_ref, topi_ref):
    x = x_ref[...]
    w = w_ref[...]
    logits = lax.dot_general(
        x, w, (((1,), (1,)), ((), ())), preferred_element_type=jnp.float32
    )
    m = jnp.max(logits, axis=1, keepdims=True)
    e = jnp.exp(logits - m)
    s = jnp.sum(e, axis=1, keepdims=True)
    probs = e / s

    iota = lax.broadcasted_iota(jnp.int32, probs.shape, 1)
    m1 = jnp.max(probs, axis=1, keepdims=True)
    i1 = jnp.min(jnp.where(probs == m1, iota, NUM_EXPERTS), axis=1, keepdims=True)
    masked = jnp.where(iota == i1, -1.0, probs)
    m2 = jnp.max(masked, axis=1, keepdims=True)
    i2 = jnp.min(jnp.where(masked == m2, iota, NUM_EXPERTS), axis=1, keepdims=True)

    denom = m1 + m2
    probs_ref[...] = probs
    topp_ref[...] = jnp.concatenate([m1 / denom, m2 / denom], axis=1)
    topi_ref[...] = jnp.concatenate([i1, i2], axis=1)


@functools.partial(jax.jit, static_argnames=("interpret",))
def kernel(x, W, interpret=False):
    n_tokens = x.shape[0]
    grid = (n_tokens // BLOCK_T,)
    probs, topp, topi = pl.pallas_call(
        _router_block,
        grid=grid,
        in_specs=[
            pl.BlockSpec((BLOCK_T, N_EMBD), lambda i: (i, 0)),
            pl.BlockSpec((NUM_EXPERTS, N_EMBD), lambda i: (0, 0)),
        ],
        out_specs=[
            pl.BlockSpec((BLOCK_T, NUM_EXPERTS), lambda i: (i, 0)),
            pl.BlockSpec((BLOCK_T, 2), lambda i: (i, 0)),
            pl.BlockSpec((BLOCK_T, 2), lambda i: (i, 0)),
        ],
        out_shape=[
            jax.ShapeDtypeStruct((n_tokens, NUM_EXPERTS), jnp.float32),
            jax.ShapeDtypeStruct((n_tokens, 2), jnp.float32),
            jax.ShapeDtypeStruct((n_tokens, 2), jnp.int32),
        ],
        interpret=interpret,
    )(x, W)
    return (topp, topi, probs)


# speedup vs baseline: 2.0126x; 1.1344x over previous
import functools

import jax
import jax.numpy as jnp
from jax import lax
from jax.experimental import pallas as pl
from jax.experimental.pallas import tpu as pltpu

N_EMBD = 768
NUM_EXPERTS = 64
N_TOKENS = 32768
BLOCK_T = 2048


def _router_block(x_ref, w_ref, probs_ref, topp_ref, topi_ref):
    x = x_ref[...]
    w = w_ref[...]
    logits = lax.dot_general(
        x, w, (((1,), (1,)), ((), ())), preferred_element_type=jnp.float32
    )
    m = jnp.max(logits, axis=1, keepdims=True)
    e = jnp.exp(logits - m)
    s = jnp.sum(e, axis=1, keepdims=True)
    probs = e / s

    iota = lax.broadcasted_iota(jnp.int32, probs.shape, 1)
    m1 = jnp.max(probs, axis=1, keepdims=True)
    i1 = jnp.min(jnp.where(probs == m1, iota, NUM_EXPERTS), axis=1, keepdims=True)
    masked = jnp.where(iota == i1, -1.0, probs)
    m2 = jnp.max(masked, axis=1, keepdims=True)
    i2 = jnp.min(jnp.where(masked == m2, iota, NUM_EXPERTS), axis=1, keepdims=True)

    denom = m1 + m2
    probs_ref[...] = probs
    topp_ref[...] = jnp.concatenate([m1 / denom, m2 / denom], axis=1)
    topi_ref[...] = jnp.concatenate([i1, i2], axis=1)


@functools.partial(jax.jit, static_argnames=("interpret",))
def kernel(x, W, interpret=False):
    n_tokens = x.shape[0]
    grid = (n_tokens // BLOCK_T,)
    probs, topp, topi = pl.pallas_call(
        _router_block,
        grid=grid,
        in_specs=[
            pl.BlockSpec((BLOCK_T, N_EMBD), lambda i: (i, 0)),
            pl.BlockSpec((NUM_EXPERTS, N_EMBD), lambda i: (0, 0)),
        ],
        out_specs=[
            pl.BlockSpec((BLOCK_T, NUM_EXPERTS), lambda i: (i, 0)),
            pl.BlockSpec((BLOCK_T, 2), lambda i: (i, 0)),
            pl.BlockSpec((BLOCK_T, 2), lambda i: (i, 0)),
        ],
        out_shape=[
            jax.ShapeDtypeStruct((n_tokens, NUM_EXPERTS), jnp.float32),
            jax.ShapeDtypeStruct((n_tokens, 2), jnp.float32),
            jax.ShapeDtypeStruct((n_tokens, 2), jnp.int32),
        ],
        interpret=interpret,
    )(x, W)
    return (topp, topi, probs)


# speedup vs baseline: 2.1162x; 1.0515x over previous
import functools

import jax
import jax.numpy as jnp
from jax import lax
from jax.experimental import pallas as pl
from jax.experimental.pallas import tpu as pltpu

N_EMBD = 768
NUM_EXPERTS = 64
N_TOKENS = 32768
BLOCK_T = 4096


def _router_block(x_ref, w_ref, probs_ref, topp_ref, topi_ref):
    x = x_ref[...]
    w = w_ref[...]
    logits = lax.dot_general(
        x, w, (((1,), (1,)), ((), ())), preferred_element_type=jnp.float32
    )
    m = jnp.max(logits, axis=1, keepdims=True)
    e = jnp.exp(logits - m)
    s = jnp.sum(e, axis=1, keepdims=True)
    probs = e / s

    iota = lax.broadcasted_iota(jnp.int32, probs.shape, 1)
    m1 = jnp.max(probs, axis=1, keepdims=True)
    i1 = jnp.min(jnp.where(probs == m1, iota, NUM_EXPERTS), axis=1, keepdims=True)
    masked = jnp.where(iota == i1, -1.0, probs)
    m2 = jnp.max(masked, axis=1, keepdims=True)
    i2 = jnp.min(jnp.where(masked == m2, iota, NUM_EXPERTS), axis=1, keepdims=True)

    denom = m1 + m2
    probs_ref[...] = probs
    topp_ref[...] = jnp.concatenate([m1 / denom, m2 / denom], axis=1)
    topi_ref[...] = jnp.concatenate([i1, i2], axis=1)


@functools.partial(jax.jit, static_argnames=("interpret",))
def kernel(x, W, interpret=False):
    n_tokens = x.shape[0]
    grid = (n_tokens // BLOCK_T,)
    probs, topp, topi = pl.pallas_call(
        _router_block,
        grid=grid,
        in_specs=[
            pl.BlockSpec((BLOCK_T, N_EMBD), lambda i: (i, 0)),
            pl.BlockSpec((NUM_EXPERTS, N_EMBD), lambda i: (0, 0)),
        ],
        out_specs=[
            pl.BlockSpec((BLOCK_T, NUM_EXPERTS), lambda i: (i, 0)),
            pl.BlockSpec((BLOCK_T, 2), lambda i: (i, 0)),
            pl.BlockSpec((BLOCK_T, 2), lambda i: (i, 0)),
        ],
        out_shape=[
            jax.ShapeDtypeStruct((n_tokens, NUM_EXPERTS), jnp.float32),
            jax.ShapeDtypeStruct((n_tokens, 2), jnp.float32),
            jax.ShapeDtypeStruct((n_tokens, 2), jnp.int32),
        ],
        interpret=interpret,
    )(x, W)
    return (topp, topi, probs)
